# baseline (speedup 1.0000x reference)
import functools

import jax
import jax.numpy as jnp
from jax import lax
from jax.experimental import pallas as pl
from jax.experimental.pallas import tpu as pltpu
from jax.experimental.pallas import tpu_sc as plsc

B, K, H, W = 2, 5000, 128, 128
N = B * K
HW = H * W
E = 320000
EPS = 1e-5

NCORES, NSUB, LANES = 2, 16, 16
NPAD = 10048
NTILES = NCORES * NSUB
KROWS = 5024
EPADDED = 327680


def _make_pool_kernel(C):
    mesh = plsc.VectorSubcoreMesh(core_axis_name="c", subcore_axis_name="s")
    NCC = C // LANES
    assert B * NCC <= NTILES
    P = 1024

    @functools.partial(
        pl.kernel,
        out_type=jax.ShapeDtypeStruct((B * KROWS, C), jnp.float32),
        mesh=mesh,
        compiler_params=pltpu.CompilerParams(use_tc_tiling_on_sc=False),
        scratch_types=[
            pltpu.VMEM((P,), jnp.int32),
            pltpu.VMEM((P, LANES), jnp.float32),
            pltpu.VMEM((K, LANES), jnp.float32),
        ],
    )
    def pool(table_hbm, labels_hbm, neginf_hbm, out_hbm, lab_v, rows_v, acc):
        cid = lax.axis_index("c")
        sid = lax.axis_index("s")
        wid = cid * NSUB + sid
        work = wid < B * NCC

        @pl.when(work)
        def _():
            b = wid // NCC
            cc = wid % NCC
            coff = pl.multiple_of(cc * LANES, LANES)
            pltpu.sync_copy(neginf_hbm, acc)
            for pc in range(HW // P):
                base = b * HW + pc * P
                pltpu.sync_copy(labels_hbm.at[pl.ds(base, P)], lab_v)
                pltpu.sync_copy(
                    table_hbm.at[pl.ds(base, P), pl.ds(coff, LANES)], rows_v)

                def ibody(i, _):
                    lv = lab_v[pl.ds(i * LANES, LANES)]
                    for r in range(LANES):
                        li = lv[r]
                        acc[li, :] = jnp.maximum(acc[li, :],
                                                 rows_v[i * LANES + r, :])
                    return 0

                lax.fori_loop(0, P // LANES, ibody, 0)

            pltpu.sync_copy(
                acc,
                out_hbm.at[pl.ds(b * KROWS, K), pl.ds(coff, LANES)])

    return pool


def _make_agg_kernel(NCH):
    mesh = plsc.VectorSubcoreMesh(core_axis_name="c", subcore_axis_name="s")
    CW = 64
    EPT = EPADDED // NSUB
    SB = 2048
    NSB = EPT // SB
    EC = 128
    NEC = SB // EC
    RING = 4
    LAG = 2
    RPT = (2 * NPAD) // NSUB
    NP = (NCH + NCORES - 1) // NCORES

    @functools.partial(
        pl.kernel,
        out_type=jax.ShapeDtypeStruct((NCH * 2 * NPAD, CW), jnp.float32),
        mesh=mesh,
        compiler_params=pltpu.CompilerParams(use_tc_tiling_on_sc=False),
        scratch_types=[
            pltpu.VMEM((SB,), jnp.int32),
            pltpu.VMEM((SB,), jnp.int32),
            pltpu.VMEM((SB,), jnp.int32),
            pltpu.VMEM((NEC, EC), jnp.int32),
            pltpu.VMEM((SB,), jnp.int32),
            [pltpu.VMEM((EC, CW), jnp.float32) for _ in range(RING)],
            [pltpu.SemaphoreType.DMA for _ in range(RING)],
            [pltpu.SemaphoreType.DMA for _ in range(RING)],
            pltpu.VMEM_SHARED((2 * NPAD, CW), jnp.float32),
        ],
    )
    def agg(xc_hbm, src_hbm, dst_hbm, sgn_hbm, zeros_hbm, out_hbm,
            src_v, dst_v, sgn_v, sidx_v, gidx_v, rows, gsem, ssem, acc):
        cid = lax.axis_index("c")
        sid = lax.axis_index("s")

        for p in range(NP):
            ch = p * NCORES + cid
            work = ch < NCH

            @pl.when(work)
            def _():
                pltpu.sync_copy(zeros_hbm, acc.at[pl.ds(sid * RPT, RPT)])

            plsc.subcore_barrier()

            @pl.when(work)
            def _():
                for sb in range(NSB):
                    e0 = sid * EPT + sb * SB
                    pltpu.sync_copy(src_hbm.at[pl.ds(e0, SB)], src_v)
                    pltpu.sync_copy(dst_hbm.at[pl.ds(e0, SB)], dst_v)
                    pltpu.sync_copy(sgn_hbm.at[pl.ds(e0, SB)], sgn_v)

                    def ibody(i, _):
                        d = dst_v[pl.ds(i * LANES, LANES)]
                        s = sgn_v[pl.ds(i * LANES, LANES)]
                        comb = d + jnp.where(s == -1, jnp.int32(NPAD),
                                             jnp.int32(0))
                        sidx_v[i // (EC // LANES),
                               pl.ds((i % (EC // LANES)) * LANES,
                                     LANES)] = comb
                        gidx_v[pl.ds(i * LANES, LANES)] = (
                            src_v[pl.ds(i * LANES, LANES)] + ch * N)
                        return 0

                    lax.fori_loop(0, SB // LANES, ibody, 0)

                    def gather(j, bi):
                        idx = gidx_v.at[pl.ds(j * EC, EC)]
                        return pltpu.async_copy(xc_hbm.at[idx], rows[bi],
                                                gsem[bi])

                    gdesc = [gather(j, j % RING) for j in range(RING)]
                    sdesc = [None] * RING
                    for j in range(NEC):
                        bi = j % RING
                        gdesc[bi].wait()
                        sdesc[bi] = pltpu.async_copy(
                            rows[bi], acc.at[sidx_v.at[j]], ssem[bi],
                            add=True)
                        jp = j - LAG
                        if jp >= 0:
                            bj = jp % RING
                            sdesc[bj].wait()
                            if jp + RING < NEC:
                                gdesc[bj] = gather(jp + RING, bj)
                    for j in range(max(NEC - LAG, 0), NEC):
                        sdesc[j % RING].wait()

            plsc.subcore_barrier()

            @pl.when(work)
            def _():
                pltpu.sync_copy(
                    acc.at[pl.ds(sid * RPT, RPT)],
                    out_hbm.at[pl.ds(ch * 2 * NPAD + sid * RPT, RPT)])

            plsc.subcore_barrier()

    return agg


RB = 2000
NBLK = N // RB


def _row_spec(*shape):
    return pl.BlockSpec((RB,) + tuple(shape),
                        lambda i: (i,) + (0,) * len(shape))


def _full_spec(shape):
    return pl.BlockSpec(tuple(shape), lambda i: (0,) * len(shape))


def _stats_update(stats_ref, y):
    @pl.when(pl.program_id(0) == 0)
    def _():
        stats_ref[...] = jnp.zeros_like(stats_ref)

    stats_ref[...] += jnp.concatenate(
        [jnp.sum(y, axis=0, keepdims=True),
         jnp.sum(y * y, axis=0, keepdims=True)], axis=0)


def _bn_relu_stats(y, stats, g, b):
    m = stats[0:1, :] * (1.0 / N)
    v = stats[1:2, :] * (1.0 / N) - m * m
    return jnp.maximum((y - m) * lax.rsqrt(v + EPS) * g + b, 0.0)


def _dot(a, b):
    return jnp.dot(a, b, preferred_element_type=jnp.float32)


def _kx0_body(small_ref, g_ref, b_ref, out_ref):
    x = small_ref[:, 0:4]
    m = jnp.mean(x, axis=0, keepdims=True)
    v = jnp.mean(x * x, axis=0, keepdims=True) - m * m
    x0 = jnp.maximum((x - m) * lax.rsqrt(v + EPS) * g_ref[:] + b_ref[:], 0.0)
    out_ref[...] = jnp.concatenate(
        [x0, small_ref[:, 4:36],
         jnp.ones((N, 1), jnp.float32),
         jnp.zeros((N, 64 - 37), jnp.float32)], axis=1)


def _ky0_body(sp_ref, sn_ref, xpad_ref, plw_ref, prw_ref, prb_ref,
              nlw_ref, nrw_ref, nrb_ref, y_ref, invc_ref, stats_ref):
    invp = 1.0 / jnp.maximum(sp_ref[:, 36:37], 1.0)
    invn = 1.0 / jnp.maximum(sn_ref[:, 36:37], 1.0)
    invc_ref[...] = jnp.concatenate(
        [invp, invn, jnp.zeros((RB, 6), jnp.float32)], axis=1)
    meanp = sp_ref[:, 0:36] * invp
    meann = sn_ref[:, 0:36] * invn
    xin = xpad_ref[:, 0:36]
    y = jnp.concatenate(
        [_dot(meanp, plw_ref[...]) + _dot(xin, prw_ref[...]) + prb_ref[:],
         _dot(meann, nlw_ref[...]) + _dot(xin, nrw_ref[...]) + nrb_ref[:]],
        axis=1)
    y_ref[...] = y
    _stats_update(stats_ref, y)


def _kz_body(y1_ref, st1_ref, skip_ref, bng_ref, bnb_ref,
             mpw_ref, mpb_ref, mnw_ref, mnb_ref, y2_ref, st2_ref):
    x = _bn_relu_stats(y1_ref[...], st1_ref[...], bng_ref[:], bnb_ref[:])
    skip = skip_ref[...]
    y2 = jnp.concatenate(
        [_dot(jnp.concatenate([x[:, 256:], skip], axis=1), mpw_ref[...])
         + mpb_ref[:],
         _dot(jnp.concatenate([x[:, 0:256], skip], axis=1), mnw_ref[...])
         + mnb_ref[:]], axis=1)
    y2_ref[...] = y2
    _stats_update(st2_ref, y2)


def _kx_body(y2_ref, st2_ref, mg_ref, mb_ref, xmc_ref):
    xm = _bn_relu_stats(y2_ref[...], st2_ref[...], mg_ref[:], mb_ref[:])
    for c in range(8):
        xmc_ref[c] = xm[:, c * 64:(c + 1) * 64]


def _ky_body(sp_ref, sn_ref, xmc_ref, invc_ref, plw_ref, prw_ref, prb_ref,
             nlw_ref, nrw_ref, nrb_ref, y_ref, stats_ref):
    invp = invc_ref[:, 0:1]
    invn = invc_ref[:, 1:2]
    x1 = jnp.concatenate([xmc_ref[c] for c in range(4)], axis=1)
    x2 = jnp.concatenate([xmc_ref[c] for c in range(4, 8)], axis=1)
    mp = sp_ref[...] * invp
    mn = sn_ref[...] * invn
    acc_p = (_dot(mp[:, 0:256], plw_ref[0:256, :])
             + _dot(mn[:, 256:512], plw_ref[256:512, :])
             + _dot(x1, prw_ref[...]) + prb_ref[:])
    acc_n = (_dot(mp[:, 256:512], nlw_ref[0:256, :])
             + _dot(mn[:, 0:256], nlw_ref[256:512, :])
             + _dot(x2, nrw_ref[...]) + nrb_ref[:])
    y = jnp.concatenate([acc_p, acc_n], axis=1)
    y_ref[...] = y
    _stats_update(stats_ref, y)


def _kf_body(y1_ref, st1_ref, bng_ref, bnb_ref, pww_ref, pwb_ref, out_ref):
    x = _bn_relu_stats(y1_ref[...], st1_ref[...], bng_ref[:], bnb_ref[:])
    out_ref[...] = jnp.maximum(_dot(x, pww_ref[...]) + pwb_ref[:], 0.0)


def _ky0_call(sp0, sn0, xpad, g0):
    return pl.pallas_call(
        _ky0_body,
        grid=(NBLK,),
        in_specs=[_row_spec(64), _row_spec(64), _row_spec(64),
                  _full_spec((36, 256)), _full_spec((36, 256)),
                  _full_spec((1, 256)),
                  _full_spec((36, 256)), _full_spec((36, 256)),
                  _full_spec((1, 256))],
        out_specs=(_row_spec(512), _row_spec(8),
                   _full_spec((2, 512))),
        out_shape=(jax.ShapeDtypeStruct((N, 512), jnp.float32),
                   jax.ShapeDtypeStruct((N, 8), jnp.float32),
                   jax.ShapeDtypeStruct((2, 512), jnp.float32)),
    )(sp0, sn0, xpad,
      g0["pos_l_w"].T, g0["pos_r_w"].T, g0["pos_r_b"].reshape(1, 256),
      g0["neg_l_w"].T, g0["neg_r_w"].T, g0["neg_r_b"].reshape(1, 256))


def _kz_call(y1, st1, skip, bng, bnb, mp, mn):
    return pl.pallas_call(
        _kz_body,
        grid=(NBLK,),
        in_specs=[_row_spec(512), _full_spec((2, 512)), _row_spec(256),
                  _full_spec((1, 512)), _full_spec((1, 512)),
                  _full_spec((512, 256)), _full_spec((1, 256)),
                  _full_spec((512, 256)), _full_spec((1, 256))],
        out_specs=(_row_spec(512), _full_spec((2, 512))),
        out_shape=(jax.ShapeDtypeStruct((N, 512), jnp.float32),
                   jax.ShapeDtypeStruct((2, 512), jnp.float32)),
    )(y1, st1, skip, bng.reshape(1, 512), bnb.reshape(1, 512),
      mp["w"].T, mp["b"].reshape(1, 256), mn["w"].T, mn["b"].reshape(1, 256))


def _kx_call(y2, st2, mp, mn):
    mg = jnp.concatenate([mp["bn_g"], mn["bn_g"]]).reshape(1, 512)
    mb = jnp.concatenate([mp["bn_b"], mn["bn_b"]]).reshape(1, 512)
    return pl.pallas_call(
        _kx_body,
        grid=(NBLK,),
        in_specs=[_row_spec(512), _full_spec((2, 512)),
                  _full_spec((1, 512)), _full_spec((1, 512))],
        out_specs=pl.BlockSpec((8, RB, 64), lambda i: (0, i, 0)),
        out_shape=jax.ShapeDtypeStruct((8, N, 64), jnp.float32),
    )(y2, st2, mg, mb)


def _ky_call(sp, sn, xmc, invc, g):
    return pl.pallas_call(
        _ky_body,
        grid=(NBLK,),
        in_specs=[_row_spec(512), _row_spec(512),
                  pl.BlockSpec((8, RB, 64), lambda i: (0, i, 0)),
                  _row_spec(8),
                  _full_spec((512, 256)), _full_spec((256, 256)),
                  _full_spec((1, 256)),
                  _full_spec((512, 256)), _full_spec((256, 256)),
                  _full_spec((1, 256))],
        out_specs=(_row_spec(512), _full_spec((2, 512))),
        out_shape=(jax.ShapeDtypeStruct((N, 512), jnp.float32),
                   jax.ShapeDtypeStruct((2, 512), jnp.float32)),
    )(sp, sn, xmc, invc,
      g["pos_l_w"].T, g["pos_r_w"].T, g["pos_r_b"].reshape(1, 256),
      g["neg_l_w"].T, g["neg_r_w"].T, g["neg_r_b"].reshape(1, 256))


def _kf_call(y1, st1, bng, bnb, pww, pwb):
    return pl.pallas_call(
        _kf_body,
        grid=(NBLK,),
        in_specs=[_row_spec(512), _full_spec((2, 512)),
                  _full_spec((1, 512)), _full_spec((1, 512)),
                  _full_spec((512, 256)), _full_spec((1, 256))],
        out_specs=_row_spec(256),
        out_shape=jax.ShapeDtypeStruct((N, 256), jnp.float32),
    )(y1, st1, bng.reshape(1, 512), bnb.reshape(1, 512),
      pww.T, pwb.reshape(1, 256))


_pool64 = _make_pool_kernel(64)
_pool256 = _make_pool_kernel(256)
_agg1 = _make_agg_kernel(1)
_agg8 = _make_agg_kernel(8)


def _unpad(x):
    return jnp.concatenate([x[0:K], x[KROWS:KROWS + K]], axis=0)


def _split_agg(agg, nch):
    a = agg.reshape(nch, 2, NPAD, 64)
    sp = a[:, 0, 0:N].transpose(1, 0, 2).reshape(N, nch * 64)
    sn = a[:, 1, 0:N].transpose(1, 0, 2).reshape(N, nch * 64)
    return sp, sn


def kernel(labels, fx, fy, skip0, skip1, skip2, edges_nn, params):
    p = params
    f32 = jnp.float32
    labels_flat = labels.reshape(B * HW)
    pidx = jnp.arange(HW, dtype=jnp.int32)
    c0 = (pidx // H).astype(f32) / (W - 1)
    c1 = (pidx % H).astype(f32) / (H - 1)
    skip0T = skip0.reshape(B, 32, HW).transpose(0, 2, 1)
    skip1T = skip1.reshape(B, 256, HW).transpose(0, 2, 1).reshape(B * HW, 256)
    skip2T = skip2.reshape(B, 256, HW).transpose(0, 2, 1).reshape(B * HW, 256)
    small = jnp.concatenate(
        [jnp.broadcast_to(c0[None, :, None], (B, HW, 1)),
         jnp.broadcast_to(c1[None, :, None], (B, HW, 1)),
         fx.reshape(B, HW, 1), fy.reshape(B, HW, 1),
         skip0T, jnp.zeros((B, HW, 28), f32)], axis=2).reshape(B * HW, 64)
    neginf = jnp.full((K, LANES), -3.0e38, f32)

    pooled_small = _unpad(_pool64(small, labels_flat, neginf))
    pooled_skip1 = _unpad(_pool256(skip1T, labels_flat, neginf))
    pooled_skip2 = _unpad(_pool256(skip2T, labels_flat, neginf))

    epad = EPADDED - E
    src = jnp.concatenate([edges_nn[0], jnp.zeros((epad,), jnp.int32)])
    dst = jnp.concatenate([edges_nn[1], jnp.full((epad,), N, jnp.int32)])
    sgn = jnp.concatenate([edges_nn[2], jnp.ones((epad,), jnp.int32)])
    zeros_rows = jnp.zeros(((2 * NPAD) // NSUB, 64), f32)

    xpad = pl.pallas_call(
        _kx0_body,
        out_shape=jax.ShapeDtypeStruct((N, 64), f32),
    )(pooled_small,
      p["bn_pre_g"].reshape(1, 4), p["bn_pre_b"].reshape(1, 4))

    agg0 = _agg1(xpad, src, dst, sgn, zeros_rows)
    sp0 = agg0[0:N]
    sn0 = agg0[NPAD:NPAD + N]

    y10, invc, st10 = _ky0_call(sp0, sn0, xpad, p["g0"])
    y20, st20 = _kz_call(y10, st10, pooled_skip1,
                         p["g0"]["bn_g"], p["g0"]["bn_b"],
                         p["m1_pos"], p["m1_neg"])
    xmc1 = _kx_call(y20, st20, p["m1_pos"], p["m1_neg"])

    agg1 = _agg8(xmc1.reshape(8 * N, 64), src, dst, sgn, zeros_rows)
    sp1, sn1 = _split_agg(agg1, 8)

    y11, st11 = _ky_call(sp1, sn1, xmc1, invc, p["g1"])
    y21, st21 = _kz_call(y11, st11, pooled_skip2,
                         p["g1"]["bn_g"], p["g1"]["bn_b"],
                         p["m2_pos"], p["m2_neg"])
    xmc2 = _kx_call(y21, st21, p["m2_pos"], p["m2_neg"])

    agg2 = _agg8(xmc2.reshape(8 * N, 64), src, dst, sgn, zeros_rows)
    sp2, sn2 = _split_agg(agg2, 8)

    y12, st12 = _ky_call(sp2, sn2, xmc2, invc, p["g2"])
    return _kf_call(y12, st12, p["g2"]["bn_g"], p["g2"]["bn_b"],
                    p["pw_w"], p["pw_b"])


# speedup vs baseline: 1.4746x; 1.4746x over previous
import functools

import jax
import jax.numpy as jnp
from jax import lax
from jax.experimental import pallas as pl
from jax.experimental.pallas import tpu as pltpu
from jax.experimental.pallas import tpu_sc as plsc

B, K, H, W = 2, 5000, 128, 128
N = B * K
HW = H * W
E = 320000
EPS = 1e-5

NCORES, NSUB, LANES = 2, 16, 16
NPAD = 10048
NTILES = NCORES * NSUB
KROWS = 5024
EPADDED = 327680


def _make_pool_kernel(C):
    mesh = plsc.VectorSubcoreMesh(core_axis_name="c", subcore_axis_name="s")
    NCC = C // LANES
    assert B * NCC <= NTILES
    P = 1024

    @functools.partial(
        pl.kernel,
        out_type=jax.ShapeDtypeStruct((B * KROWS, C), jnp.float32),
        mesh=mesh,
        compiler_params=pltpu.CompilerParams(use_tc_tiling_on_sc=False),
        scratch_types=[
            pltpu.VMEM((P,), jnp.int32),
            pltpu.VMEM((P, LANES), jnp.float32),
            pltpu.VMEM((K, LANES), jnp.float32),
        ],
    )
    def pool(table_hbm, labels_hbm, neginf_hbm, out_hbm, lab_v, rows_v, acc):
        cid = lax.axis_index("c")
        sid = lax.axis_index("s")
        wid = cid * NSUB + sid
        work = wid < B * NCC

        @pl.when(work)
        def _():
            b = wid // NCC
            cc = wid % NCC
            coff = pl.multiple_of(cc * LANES, LANES)
            pltpu.sync_copy(neginf_hbm, acc)
            for pc in range(HW // P):
                base = b * HW + pc * P
                pltpu.sync_copy(labels_hbm.at[pl.ds(base, P)], lab_v)
                pltpu.sync_copy(
                    table_hbm.at[pl.ds(base, P), pl.ds(coff, LANES)], rows_v)

                def ibody(i, _):
                    lv = lab_v[pl.ds(i * LANES, LANES)]
                    for r in range(LANES):
                        li = lv[r]
                        acc[li, :] = jnp.maximum(acc[li, :],
                                                 rows_v[i * LANES + r, :])
                    return 0

                lax.fori_loop(0, P // LANES, ibody, 0)

            pltpu.sync_copy(
                acc,
                out_hbm.at[pl.ds(b * KROWS, K), pl.ds(coff, LANES)])

    return pool


def _make_agg_kernel(NCH, CW):
    mesh = plsc.VectorSubcoreMesh(core_axis_name="c", subcore_axis_name="s")
    EPT = EPADDED // NSUB
    SB = 2048
    NSB = EPT // SB
    EC = 128
    NEC = SB // EC
    RING = 4
    LAG = 2
    RPT = (2 * NPAD) // NSUB
    NP = (NCH + NCORES - 1) // NCORES

    @functools.partial(
        pl.kernel,
        out_type=jax.ShapeDtypeStruct((NCH * 2 * NPAD, CW), jnp.bfloat16),
        mesh=mesh,
        compiler_params=pltpu.CompilerParams(use_tc_tiling_on_sc=False),
        scratch_types=[
            pltpu.VMEM((SB,), jnp.int32),
            pltpu.VMEM((SB,), jnp.int32),
            pltpu.VMEM((SB,), jnp.int32),
            pltpu.VMEM((NEC, EC), jnp.int32),
            pltpu.VMEM((SB,), jnp.int32),
            [pltpu.VMEM((EC, CW), jnp.bfloat16) for _ in range(RING)],
            [pltpu.SemaphoreType.DMA for _ in range(RING)],
            [pltpu.SemaphoreType.DMA for _ in range(RING)],
            pltpu.VMEM_SHARED((2 * NPAD, CW), jnp.bfloat16),
        ],
    )
    def agg(xc_hbm, src_hbm, dst_hbm, sgn_hbm, zeros_hbm, out_hbm,
            src_v, dst_v, sgn_v, sidx_v, gidx_v, rows, gsem, ssem, acc):
        cid = lax.axis_index("c")
        sid = lax.axis_index("s")

        for p in range(NP):
            ch = p * NCORES + cid
            work = ch < NCH

            @pl.when(work)
            def _():
                pltpu.sync_copy(zeros_hbm, acc.at[pl.ds(sid * RPT, RPT)])

            plsc.subcore_barrier()

            @pl.when(work)
            def _():
                for sb in range(NSB):
                    e0 = sid * EPT + sb * SB
                    pltpu.sync_copy(src_hbm.at[pl.ds(e0, SB)], src_v)
                    pltpu.sync_copy(dst_hbm.at[pl.ds(e0, SB)], dst_v)
                    pltpu.sync_copy(sgn_hbm.at[pl.ds(e0, SB)], sgn_v)

                    def ibody(i, _):
                        d = dst_v[pl.ds(i * LANES, LANES)]
                        s = sgn_v[pl.ds(i * LANES, LANES)]
                        comb = d + jnp.where(s == -1, jnp.int32(NPAD),
                                             jnp.int32(0))
                        sidx_v[i // (EC // LANES),
                               pl.ds((i % (EC // LANES)) * LANES,
                                     LANES)] = comb
                        gidx_v[pl.ds(i * LANES, LANES)] = (
                            src_v[pl.ds(i * LANES, LANES)] + ch * N)
                        return 0

                    lax.fori_loop(0, SB // LANES, ibody, 0)

                    def gather(j, bi):
                        idx = gidx_v.at[pl.ds(j * EC, EC)]
                        return pltpu.async_copy(xc_hbm.at[idx], rows[bi],
                                                gsem[bi])

                    gdesc = [gather(j, j % RING) for j in range(RING)]
                    sdesc = [None] * RING
                    for j in range(NEC):
                        bi = j % RING
                        gdesc[bi].wait()
                        sdesc[bi] = pltpu.async_copy(
                            rows[bi], acc.at[sidx_v.at[j]], ssem[bi],
                            add=True)
                        jp = j - LAG
                        if jp >= 0:
                            bj = jp % RING
                            sdesc[bj].wait()
                            if jp + RING < NEC:
                                gdesc[bj] = gather(jp + RING, bj)
                    for j in range(max(NEC - LAG, 0), NEC):
                        sdesc[j % RING].wait()

            plsc.subcore_barrier()

            @pl.when(work)
            def _():
                pltpu.sync_copy(
                    acc.at[pl.ds(sid * RPT, RPT)],
                    out_hbm.at[pl.ds(ch * 2 * NPAD + sid * RPT, RPT)])

            plsc.subcore_barrier()

    return agg


RB = 2000
NBLK = N // RB


def _row_spec(*shape):
    return pl.BlockSpec((RB,) + tuple(shape),
                        lambda i: (i,) + (0,) * len(shape))


def _full_spec(shape):
    return pl.BlockSpec(tuple(shape), lambda i: (0,) * len(shape))


def _stats_update(stats_ref, y):
    @pl.when(pl.program_id(0) == 0)
    def _():
        stats_ref[...] = jnp.zeros_like(stats_ref)

    stats_ref[...] += jnp.concatenate(
        [jnp.sum(y, axis=0, keepdims=True),
         jnp.sum(y * y, axis=0, keepdims=True)], axis=0)


def _bn_relu_stats(y, stats, g, b):
    m = stats[0:1, :] * (1.0 / N)
    v = stats[1:2, :] * (1.0 / N) - m * m
    return jnp.maximum((y - m) * lax.rsqrt(v + EPS) * g + b, 0.0)


def _dot(a, b):
    return jnp.dot(a, b, preferred_element_type=jnp.float32)


def _kx0_body(small_ref, g_ref, b_ref, out_ref, outb_ref):
    x = small_ref[:, 0:4]
    m = jnp.mean(x, axis=0, keepdims=True)
    v = jnp.mean(x * x, axis=0, keepdims=True) - m * m
    x0 = jnp.maximum((x - m) * lax.rsqrt(v + EPS) * g_ref[:] + b_ref[:], 0.0)
    xp = jnp.concatenate(
        [x0, small_ref[:, 4:36],
         jnp.ones((N, 1), jnp.float32),
         jnp.zeros((N, 64 - 37), jnp.float32)], axis=1)
    out_ref[...] = xp
    outb_ref[...] = xp.astype(jnp.bfloat16)


def _ky0_body(sp_ref, sn_ref, xpad_ref, plw_ref, prw_ref, prb_ref,
              nlw_ref, nrw_ref, nrb_ref, y_ref, invc_ref, stats_ref):
    invp = 1.0 / jnp.maximum(sp_ref[:, 36:37].astype(jnp.float32), 1.0)
    invn = 1.0 / jnp.maximum(sn_ref[:, 36:37].astype(jnp.float32), 1.0)
    invc_ref[...] = jnp.concatenate(
        [invp, invn, jnp.zeros((RB, 6), jnp.float32)], axis=1)
    meanp = sp_ref[:, 0:36].astype(jnp.float32) * invp
    meann = sn_ref[:, 0:36].astype(jnp.float32) * invn
    xin = xpad_ref[:, 0:36]
    y = jnp.concatenate(
        [_dot(meanp, plw_ref[...]) + _dot(xin, prw_ref[...]) + prb_ref[:],
         _dot(meann, nlw_ref[...]) + _dot(xin, nrw_ref[...]) + nrb_ref[:]],
        axis=1)
    y_ref[...] = y
    _stats_update(stats_ref, y)


def _kz_body(y1_ref, st1_ref, skip_ref, bng_ref, bnb_ref,
             mpw_ref, mpb_ref, mnw_ref, mnb_ref, y2_ref, st2_ref):
    x = _bn_relu_stats(y1_ref[...], st1_ref[...], bng_ref[:], bnb_ref[:])
    skip = skip_ref[...]
    y2 = jnp.concatenate(
        [_dot(jnp.concatenate([x[:, 256:], skip], axis=1), mpw_ref[...])
         + mpb_ref[:],
         _dot(jnp.concatenate([x[:, 0:256], skip], axis=1), mnw_ref[...])
         + mnb_ref[:]], axis=1)
    y2_ref[...] = y2
    _stats_update(st2_ref, y2)


def _kx_body(y2_ref, st2_ref, mg_ref, mb_ref, xmc_ref, xcb_ref):
    xm = _bn_relu_stats(y2_ref[...], st2_ref[...], mg_ref[:], mb_ref[:])
    for c in range(8):
        xmc_ref[c] = xm[:, c * 64:(c + 1) * 64]
    xmb = xm.astype(jnp.bfloat16)
    for c in range(4):
        xcb_ref[c] = xmb[:, c * 128:(c + 1) * 128]


def _ky_body(sp_ref, sn_ref, xmc_ref, invc_ref, plw_ref, prw_ref, prb_ref,
             nlw_ref, nrw_ref, nrb_ref, y_ref, stats_ref):
    invp = invc_ref[:, 0:1]
    invn = invc_ref[:, 1:2]
    x1 = jnp.concatenate([xmc_ref[c] for c in range(4)], axis=1)
    x2 = jnp.concatenate([xmc_ref[c] for c in range(4, 8)], axis=1)
    mp = sp_ref[...].astype(jnp.float32) * invp
    mn = sn_ref[...].astype(jnp.float32) * invn
    acc_p = (_dot(mp[:, 0:256], plw_ref[0:256, :])
             + _dot(mn[:, 256:512], plw_ref[256:512, :])
             + _dot(x1, prw_ref[...]) + prb_ref[:])
    acc_n = (_dot(mp[:, 256:512], nlw_ref[0:256, :])
             + _dot(mn[:, 0:256], nlw_ref[256:512, :])
             + _dot(x2, nrw_ref[...]) + nrb_ref[:])
    y = jnp.concatenate([acc_p, acc_n], axis=1)
    y_ref[...] = y
    _stats_update(stats_ref, y)


def _kf_body(y1_ref, st1_ref, bng_ref, bnb_ref, pww_ref, pwb_ref, out_ref):
    x = _bn_relu_stats(y1_ref[...], st1_ref[...], bng_ref[:], bnb_ref[:])
    out_ref[...] = jnp.maximum(_dot(x, pww_ref[...]) + pwb_ref[:], 0.0)


def _ky0_call(sp0, sn0, xpad, g0):
    return pl.pallas_call(
        _ky0_body,
        grid=(NBLK,),
        in_specs=[_row_spec(64), _row_spec(64), _row_spec(64),
                  _full_spec((36, 256)), _full_spec((36, 256)),
                  _full_spec((1, 256)),
                  _full_spec((36, 256)), _full_spec((36, 256)),
                  _full_spec((1, 256))],
        out_specs=(_row_spec(512), _row_spec(8),
                   _full_spec((2, 512))),
        out_shape=(jax.ShapeDtypeStruct((N, 512), jnp.float32),
                   jax.ShapeDtypeStruct((N, 8), jnp.float32),
                   jax.ShapeDtypeStruct((2, 512), jnp.float32)),
    )(sp0, sn0, xpad,
      g0["pos_l_w"].T, g0["pos_r_w"].T, g0["pos_r_b"].reshape(1, 256),
      g0["neg_l_w"].T, g0["neg_r_w"].T, g0["neg_r_b"].reshape(1, 256))


def _kz_call(y1, st1, skip, bng, bnb, mp, mn):
    return pl.pallas_call(
        _kz_body,
        grid=(NBLK,),
        in_specs=[_row_spec(512), _full_spec((2, 512)), _row_spec(256),
                  _full_spec((1, 512)), _full_spec((1, 512)),
                  _full_spec((512, 256)), _full_spec((1, 256)),
                  _full_spec((512, 256)), _full_spec((1, 256))],
        out_specs=(_row_spec(512), _full_spec((2, 512))),
        out_shape=(jax.ShapeDtypeStruct((N, 512), jnp.float32),
                   jax.ShapeDtypeStruct((2, 512), jnp.float32)),
    )(y1, st1, skip, bng.reshape(1, 512), bnb.reshape(1, 512),
      mp["w"].T, mp["b"].reshape(1, 256), mn["w"].T, mn["b"].reshape(1, 256))


def _kx_call(y2, st2, mp, mn):
    mg = jnp.concatenate([mp["bn_g"], mn["bn_g"]]).reshape(1, 512)
    mb = jnp.concatenate([mp["bn_b"], mn["bn_b"]]).reshape(1, 512)
    return pl.pallas_call(
        _kx_body,
        grid=(NBLK,),
        in_specs=[_row_spec(512), _full_spec((2, 512)),
                  _full_spec((1, 512)), _full_spec((1, 512))],
        out_specs=(pl.BlockSpec((8, RB, 64), lambda i: (0, i, 0)),
                   pl.BlockSpec((4, RB, 128), lambda i: (0, i, 0))),
        out_shape=(jax.ShapeDtypeStruct((8, N, 64), jnp.float32),
                   jax.ShapeDtypeStruct((4, N, 128), jnp.bfloat16)),
    )(y2, st2, mg, mb)


def _ky_call(sp, sn, xmc, invc, g):
    return pl.pallas_call(
        _ky_body,
        grid=(NBLK,),
        in_specs=[_row_spec(512), _row_spec(512),
                  pl.BlockSpec((8, RB, 64), lambda i: (0, i, 0)),
                  _row_spec(8),
                  _full_spec((512, 256)), _full_spec((256, 256)),
                  _full_spec((1, 256)),
                  _full_spec((512, 256)), _full_spec((256, 256)),
                  _full_spec((1, 256))],
        out_specs=(_row_spec(512), _full_spec((2, 512))),
        out_shape=(jax.ShapeDtypeStruct((N, 512), jnp.float32),
                   jax.ShapeDtypeStruct((2, 512), jnp.float32)),
    )(sp, sn, xmc, invc,
      g["pos_l_w"].T, g["pos_r_w"].T, g["pos_r_b"].reshape(1, 256),
      g["neg_l_w"].T, g["neg_r_w"].T, g["neg_r_b"].reshape(1, 256))


def _kf_call(y1, st1, bng, bnb, pww, pwb):
    return pl.pallas_call(
        _kf_body,
        grid=(NBLK,),
        in_specs=[_row_spec(512), _full_spec((2, 512)),
                  _full_spec((1, 512)), _full_spec((1, 512)),
                  _full_spec((512, 256)), _full_spec((1, 256))],
        out_specs=_row_spec(256),
        out_shape=jax.ShapeDtypeStruct((N, 256), jnp.float32),
    )(y1, st1, bng.reshape(1, 512), bnb.reshape(1, 512),
      pww.T, pwb.reshape(1, 256))


_pool64 = _make_pool_kernel(64)
_pool256 = _make_pool_kernel(256)
_agg1 = _make_agg_kernel(1, 64)
_agg4 = _make_agg_kernel(4, 128)


def _unpad(x):
    return jnp.concatenate([x[0:K], x[KROWS:KROWS + K]], axis=0)


def _split_agg(agg, nch, cw):
    a = agg.reshape(nch, 2, NPAD, cw)
    sp = a[:, 0, 0:N].transpose(1, 0, 2).reshape(N, nch * cw)
    sn = a[:, 1, 0:N].transpose(1, 0, 2).reshape(N, nch * cw)
    return sp, sn


def kernel(labels, fx, fy, skip0, skip1, skip2, edges_nn, params):
    p = params
    f32 = jnp.float32
    labels_flat = labels.reshape(B * HW)
    pidx = jnp.arange(HW, dtype=jnp.int32)
    c0 = (pidx // H).astype(f32) / (W - 1)
    c1 = (pidx % H).astype(f32) / (H - 1)
    skip0T = skip0.reshape(B, 32, HW).transpose(0, 2, 1)
    skip1T = skip1.reshape(B, 256, HW).transpose(0, 2, 1).reshape(B * HW, 256)
    skip2T = skip2.reshape(B, 256, HW).transpose(0, 2, 1).reshape(B * HW, 256)
    small = jnp.concatenate(
        [jnp.broadcast_to(c0[None, :, None], (B, HW, 1)),
         jnp.broadcast_to(c1[None, :, None], (B, HW, 1)),
         fx.reshape(B, HW, 1), fy.reshape(B, HW, 1),
         skip0T, jnp.zeros((B, HW, 28), f32)], axis=2).reshape(B * HW, 64)
    neginf = jnp.full((K, LANES), -3.0e38, f32)

    pooled_small = _unpad(_pool64(small, labels_flat, neginf))
    pooled_skip1 = _unpad(_pool256(skip1T, labels_flat, neginf))
    pooled_skip2 = _unpad(_pool256(skip2T, labels_flat, neginf))

    epad = EPADDED - E
    src = jnp.concatenate([edges_nn[0], jnp.zeros((epad,), jnp.int32)])
    dst = jnp.concatenate([edges_nn[1], jnp.full((epad,), N, jnp.int32)])
    sgn = jnp.concatenate([edges_nn[2], jnp.ones((epad,), jnp.int32)])
    zeros64 = jnp.zeros(((2 * NPAD) // NSUB, 64), jnp.bfloat16)
    zeros128 = jnp.zeros(((2 * NPAD) // NSUB, 128), jnp.bfloat16)

    xpad, xpadb = pl.pallas_call(
        _kx0_body,
        out_shape=(jax.ShapeDtypeStruct((N, 64), f32),
                   jax.ShapeDtypeStruct((N, 64), jnp.bfloat16)),
    )(pooled_small,
      p["bn_pre_g"].reshape(1, 4), p["bn_pre_b"].reshape(1, 4))

    agg0 = _agg1(xpadb, src, dst, sgn, zeros64)
    sp0 = agg0[0:N]
    sn0 = agg0[NPAD:NPAD + N]

    y10, invc, st10 = _ky0_call(sp0, sn0, xpad, p["g0"])
    y20, st20 = _kz_call(y10, st10, pooled_skip1,
                         p["g0"]["bn_g"], p["g0"]["bn_b"],
                         p["m1_pos"], p["m1_neg"])
    xmc1, xcb1 = _kx_call(y20, st20, p["m1_pos"], p["m1_neg"])

    agg1 = _agg4(xcb1.reshape(4 * N, 128), src, dst, sgn, zeros128)
    sp1, sn1 = _split_agg(agg1, 4, 128)

    y11, st11 = _ky_call(sp1, sn1, xmc1, invc, p["g1"])
    y21, st21 = _kz_call(y11, st11, pooled_skip2,
                         p["g1"]["bn_g"], p["g1"]["bn_b"],
                         p["m2_pos"], p["m2_neg"])
    xmc2, xcb2 = _kx_call(y21, st21, p["m2_pos"], p["m2_neg"])

    agg2 = _agg4(xcb2.reshape(4 * N, 128), src, dst, sgn, zeros128)
    sp2, sn2 = _split_agg(agg2, 4, 128)

    y12, st12 = _ky_call(sp2, sn2, xmc2, invc, p["g2"])
    return _kf_call(y12, st12, p["g2"]["bn_g"], p["g2"]["bn_b"],
                    p["pw_w"], p["pw_b"])


# speedup vs baseline: 1.4755x; 1.0006x over previous
import functools

import jax
import jax.numpy as jnp
from jax import lax
from jax.experimental import pallas as pl
from jax.experimental.pallas import tpu as pltpu
from jax.experimental.pallas import tpu_sc as plsc

B, K, H, W = 2, 5000, 128, 128
N = B * K
HW = H * W
E = 320000
EPS = 1e-5

NCORES, NSUB, LANES = 2, 16, 16
NPAD = 10048
NTILES = NCORES * NSUB
KROWS = 5024
EPADDED = 327680


def _make_pool_kernel(C):
    mesh = plsc.VectorSubcoreMesh(core_axis_name="c", subcore_axis_name="s")
    NCC = C // LANES
    assert B * NCC <= NTILES
    P = 1024

    @functools.partial(
        pl.kernel,
        out_type=jax.ShapeDtypeStruct((B * KROWS, C), jnp.float32),
        mesh=mesh,
        compiler_params=pltpu.CompilerParams(use_tc_tiling_on_sc=False),
        scratch_types=[
            pltpu.VMEM((P,), jnp.int32),
            pltpu.VMEM((P, LANES), jnp.float32),
            pltpu.VMEM((K, LANES), jnp.float32),
        ],
    )
    def pool(table_hbm, labels_hbm, neginf_hbm, out_hbm, lab_v, rows_v, acc):
        cid = lax.axis_index("c")
        sid = lax.axis_index("s")
        wid = cid * NSUB + sid
        work = wid < B * NCC

        @pl.when(work)
        def _():
            b = wid // NCC
            cc = wid % NCC
            coff = pl.multiple_of(cc * LANES, LANES)
            pltpu.sync_copy(neginf_hbm, acc)
            for pc in range(HW // P):
                base = b * HW + pc * P
                pltpu.sync_copy(labels_hbm.at[pl.ds(base, P)], lab_v)
                pltpu.sync_copy(
                    table_hbm.at[pl.ds(base, P), pl.ds(coff, LANES)], rows_v)

                def ibody(i, _):
                    lv = lab_v[pl.ds(i * LANES, LANES)]
                    for r in range(LANES):
                        li = lv[r]
                        acc[li, :] = jnp.maximum(acc[li, :],
                                                 rows_v[i * LANES + r, :])
                    return 0

                lax.fori_loop(0, P // LANES, ibody, 0)

            pltpu.sync_copy(
                acc,
                out_hbm.at[pl.ds(b * KROWS, K), pl.ds(coff, LANES)])

    return pool


def _make_agg_kernel(NCH, CW, split=False):
    mesh = plsc.VectorSubcoreMesh(core_axis_name="c", subcore_axis_name="s")
    NW = NSUB * (NCORES if split else 1)
    EPT = EPADDED // NW
    SB = 2048
    NSB = EPT // SB
    EC = 128
    NEC = SB // EC
    RING = 4
    LAG = 2
    RPT = (2 * NPAD) // NSUB
    NP = (NCH + NCORES - 1) // NCORES

    @functools.partial(
        pl.kernel,
        out_type=jax.ShapeDtypeStruct(((2 if split else NCH) * 2 * NPAD, CW), jnp.bfloat16),
        mesh=mesh,
        compiler_params=pltpu.CompilerParams(use_tc_tiling_on_sc=False),
        scratch_types=[
            pltpu.VMEM((SB,), jnp.int32),
            pltpu.VMEM((SB,), jnp.int32),
            pltpu.VMEM((SB,), jnp.int32),
            pltpu.VMEM((NEC, EC), jnp.int32),
            pltpu.VMEM((SB,), jnp.int32),
            [pltpu.VMEM((EC, CW), jnp.bfloat16) for _ in range(RING)],
            [pltpu.SemaphoreType.DMA for _ in range(RING)],
            [pltpu.SemaphoreType.DMA for _ in range(RING)],
            pltpu.VMEM_SHARED((2 * NPAD, CW), jnp.bfloat16),
        ],
    )
    def agg(xc_hbm, src_hbm, dst_hbm, sgn_hbm, zeros_hbm, out_hbm,
            src_v, dst_v, sgn_v, sidx_v, gidx_v, rows, gsem, ssem, acc):
        cid = lax.axis_index("c")
        sid = lax.axis_index("s")

        for p in range(NP):
            ch = jnp.int32(0) if split else p * NCORES + cid
            work = True if split else ch < NCH
            wid = cid * NSUB + sid if split else sid
            obase = cid * 2 * NPAD if split else ch * 2 * NPAD

            @pl.when(work)
            def _():
                pltpu.sync_copy(zeros_hbm, acc.at[pl.ds(sid * RPT, RPT)])

            plsc.subcore_barrier()

            @pl.when(work)
            def _():
                for sb in range(NSB):
                    e0 = wid * EPT + sb * SB
                    pltpu.sync_copy(src_hbm.at[pl.ds(e0, SB)], src_v)
                    pltpu.sync_copy(dst_hbm.at[pl.ds(e0, SB)], dst_v)
                    pltpu.sync_copy(sgn_hbm.at[pl.ds(e0, SB)], sgn_v)

                    def ibody(i, _):
                        d = dst_v[pl.ds(i * LANES, LANES)]
                        s = sgn_v[pl.ds(i * LANES, LANES)]
                        comb = d + jnp.where(s == -1, jnp.int32(NPAD),
                                             jnp.int32(0))
                        sidx_v[i // (EC // LANES),
                               pl.ds((i % (EC // LANES)) * LANES,
                                     LANES)] = comb
                        gidx_v[pl.ds(i * LANES, LANES)] = (
                            src_v[pl.ds(i * LANES, LANES)] + ch * N)
                        return 0

                    lax.fori_loop(0, SB // LANES, ibody, 0)

                    def gather(j, bi):
                        idx = gidx_v.at[pl.ds(j * EC, EC)]
                        return pltpu.async_copy(xc_hbm.at[idx], rows[bi],
                                                gsem[bi])

                    gdesc = [gather(j, j % RING) for j in range(RING)]
                    sdesc = [None] * RING
                    for j in range(NEC):
                        bi = j % RING
                        gdesc[bi].wait()
                        sdesc[bi] = pltpu.async_copy(
                            rows[bi], acc.at[sidx_v.at[j]], ssem[bi],
                            add=True)
                        jp = j - LAG
                        if jp >= 0:
                            bj = jp % RING
                            sdesc[bj].wait()
                            if jp + RING < NEC:
                                gdesc[bj] = gather(jp + RING, bj)
                    for j in range(max(NEC - LAG, 0), NEC):
                        sdesc[j % RING].wait()

            plsc.subcore_barrier()

            @pl.when(work)
            def _():
                pltpu.sync_copy(
                    acc.at[pl.ds(sid * RPT, RPT)],
                    out_hbm.at[pl.ds(obase + sid * RPT, RPT)])

            plsc.subcore_barrier()

    return agg


RB = 2000
NBLK = N // RB


def _row_spec(*shape):
    return pl.BlockSpec((RB,) + tuple(shape),
                        lambda i: (i,) + (0,) * len(shape))


def _full_spec(shape):
    return pl.BlockSpec(tuple(shape), lambda i: (0,) * len(shape))


def _stats_update(stats_ref, y):
    @pl.when(pl.program_id(0) == 0)
    def _():
        stats_ref[...] = jnp.zeros_like(stats_ref)

    stats_ref[...] += jnp.concatenate(
        [jnp.sum(y, axis=0, keepdims=True),
         jnp.sum(y * y, axis=0, keepdims=True)], axis=0)


def _bn_relu_stats(y, stats, g, b):
    m = stats[0:1, :] * (1.0 / N)
    v = stats[1:2, :] * (1.0 / N) - m * m
    return jnp.maximum((y - m) * lax.rsqrt(v + EPS) * g + b, 0.0)


def _dot(a, b):
    return jnp.dot(a, b, preferred_element_type=jnp.float32)


def _kx0_body(small_ref, g_ref, b_ref, out_ref, outb_ref):
    x = small_ref[:, 0:4]
    m = jnp.mean(x, axis=0, keepdims=True)
    v = jnp.mean(x * x, axis=0, keepdims=True) - m * m
    x0 = jnp.maximum((x - m) * lax.rsqrt(v + EPS) * g_ref[:] + b_ref[:], 0.0)
    xp = jnp.concatenate(
        [x0, small_ref[:, 4:36],
         jnp.ones((N, 1), jnp.float32),
         jnp.zeros((N, 64 - 37), jnp.float32)], axis=1)
    out_ref[...] = xp
    outb_ref[...] = xp.astype(jnp.bfloat16)


def _ky0_body(spa_ref, sna_ref, spb_ref, snb_ref, xpad_ref,
              plw_ref, prw_ref, prb_ref,
              nlw_ref, nrw_ref, nrb_ref, y_ref, invc_ref, stats_ref):
    sp = spa_ref[...].astype(jnp.float32) + spb_ref[...].astype(jnp.float32)
    sn = sna_ref[...].astype(jnp.float32) + snb_ref[...].astype(jnp.float32)
    invp = 1.0 / jnp.maximum(sp[:, 36:37], 1.0)
    invn = 1.0 / jnp.maximum(sn[:, 36:37], 1.0)
    invc_ref[...] = jnp.concatenate(
        [invp, invn, jnp.zeros((RB, 6), jnp.float32)], axis=1)
    meanp = sp[:, 0:36] * invp
    meann = sn[:, 0:36] * invn
    xin = xpad_ref[:, 0:36]
    y = jnp.concatenate(
        [_dot(meanp, plw_ref[...]) + _dot(xin, prw_ref[...]) + prb_ref[:],
         _dot(meann, nlw_ref[...]) + _dot(xin, nrw_ref[...]) + nrb_ref[:]],
        axis=1)
    y_ref[...] = y
    _stats_update(stats_ref, y)


def _kz_body(y1_ref, st1_ref, skip_ref, bng_ref, bnb_ref,
             mpw_ref, mpb_ref, mnw_ref, mnb_ref, y2_ref, st2_ref):
    x = _bn_relu_stats(y1_ref[...], st1_ref[...], bng_ref[:], bnb_ref[:])
    skip = skip_ref[...]
    y2 = jnp.concatenate(
        [_dot(jnp.concatenate([x[:, 256:], skip], axis=1), mpw_ref[...])
         + mpb_ref[:],
         _dot(jnp.concatenate([x[:, 0:256], skip], axis=1), mnw_ref[...])
         + mnb_ref[:]], axis=1)
    y2_ref[...] = y2
    _stats_update(st2_ref, y2)


def _kx_body(y2_ref, st2_ref, mg_ref, mb_ref, xmc_ref, xcb_ref):
    xm = _bn_relu_stats(y2_ref[...], st2_ref[...], mg_ref[:], mb_ref[:])
    for c in range(8):
        xmc_ref[c] = xm[:, c * 64:(c + 1) * 64]
    xmb = xm.astype(jnp.bfloat16)
    for c in range(4):
        xcb_ref[c] = xmb[:, c * 128:(c + 1) * 128]


def _ky_body(sp_ref, sn_ref, xmc_ref, invc_ref, plw_ref, prw_ref, prb_ref,
             nlw_ref, nrw_ref, nrb_ref, y_ref, stats_ref):
    invp = invc_ref[:, 0:1]
    invn = invc_ref[:, 1:2]
    x1 = jnp.concatenate([xmc_ref[c] for c in range(4)], axis=1)
    x2 = jnp.concatenate([xmc_ref[c] for c in range(4, 8)], axis=1)
    mp = sp_ref[...].astype(jnp.float32) * invp
    mn = sn_ref[...].astype(jnp.float32) * invn
    acc_p = (_dot(mp[:, 0:256], plw_ref[0:256, :])
             + _dot(mn[:, 256:512], plw_ref[256:512, :])
             + _dot(x1, prw_ref[...]) + prb_ref[:])
    acc_n = (_dot(mp[:, 256:512], nlw_ref[0:256, :])
             + _dot(mn[:, 0:256], nlw_ref[256:512, :])
             + _dot(x2, nrw_ref[...]) + nrb_ref[:])
    y = jnp.concatenate([acc_p, acc_n], axis=1)
    y_ref[...] = y
    _stats_update(stats_ref, y)


def _kf_body(y1_ref, st1_ref, bng_ref, bnb_ref, pww_ref, pwb_ref, out_ref):
    x = _bn_relu_stats(y1_ref[...], st1_ref[...], bng_ref[:], bnb_ref[:])
    out_ref[...] = jnp.maximum(_dot(x, pww_ref[...]) + pwb_ref[:], 0.0)


def _ky0_call(sp0, sn0, sp0b, sn0b, xpad, g0):
    return pl.pallas_call(
        _ky0_body,
        grid=(NBLK,),
        in_specs=[_row_spec(64), _row_spec(64), _row_spec(64),
                  _row_spec(64), _row_spec(64),
                  _full_spec((36, 256)), _full_spec((36, 256)),
                  _full_spec((1, 256)),
                  _full_spec((36, 256)), _full_spec((36, 256)),
                  _full_spec((1, 256))],
        out_specs=(_row_spec(512), _row_spec(8),
                   _full_spec((2, 512))),
        out_shape=(jax.ShapeDtypeStruct((N, 512), jnp.float32),
                   jax.ShapeDtypeStruct((N, 8), jnp.float32),
                   jax.ShapeDtypeStruct((2, 512), jnp.float32)),
    )(sp0, sn0, sp0b, sn0b, xpad,
      g0["pos_l_w"].T, g0["pos_r_w"].T, g0["pos_r_b"].reshape(1, 256),
      g0["neg_l_w"].T, g0["neg_r_w"].T, g0["neg_r_b"].reshape(1, 256))


def _kz_call(y1, st1, skip, bng, bnb, mp, mn):
    return pl.pallas_call(
        _kz_body,
        grid=(NBLK,),
        in_specs=[_row_spec(512), _full_spec((2, 512)), _row_spec(256),
                  _full_spec((1, 512)), _full_spec((1, 512)),
                  _full_spec((512, 256)), _full_spec((1, 256)),
                  _full_spec((512, 256)), _full_spec((1, 256))],
        out_specs=(_row_spec(512), _full_spec((2, 512))),
        out_shape=(jax.ShapeDtypeStruct((N, 512), jnp.float32),
                   jax.ShapeDtypeStruct((2, 512), jnp.float32)),
    )(y1, st1, skip, bng.reshape(1, 512), bnb.reshape(1, 512),
      mp["w"].T, mp["b"].reshape(1, 256), mn["w"].T, mn["b"].reshape(1, 256))


def _kx_call(y2, st2, mp, mn):
    mg = jnp.concatenate([mp["bn_g"], mn["bn_g"]]).reshape(1, 512)
    mb = jnp.concatenate([mp["bn_b"], mn["bn_b"]]).reshape(1, 512)
    return pl.pallas_call(
        _kx_body,
        grid=(NBLK,),
        in_specs=[_row_spec(512), _full_spec((2, 512)),
                  _full_spec((1, 512)), _full_spec((1, 512))],
        out_specs=(pl.BlockSpec((8, RB, 64), lambda i: (0, i, 0)),
                   pl.BlockSpec((4, RB, 128), lambda i: (0, i, 0))),
        out_shape=(jax.ShapeDtypeStruct((8, N, 64), jnp.float32),
                   jax.ShapeDtypeStruct((4, N, 128), jnp.bfloat16)),
    )(y2, st2, mg, mb)


def _ky_call(sp, sn, xmc, invc, g):
    return pl.pallas_call(
        _ky_body,
        grid=(NBLK,),
        in_specs=[_row_spec(512), _row_spec(512),
                  pl.BlockSpec((8, RB, 64), lambda i: (0, i, 0)),
                  _row_spec(8),
                  _full_spec((512, 256)), _full_spec((256, 256)),
                  _full_spec((1, 256)),
                  _full_spec((512, 256)), _full_spec((256, 256)),
                  _full_spec((1, 256))],
        out_specs=(_row_spec(512), _full_spec((2, 512))),
        out_shape=(jax.ShapeDtypeStruct((N, 512), jnp.float32),
                   jax.ShapeDtypeStruct((2, 512), jnp.float32)),
    )(sp, sn, xmc, invc,
      g["pos_l_w"].T, g["pos_r_w"].T, g["pos_r_b"].reshape(1, 256),
      g["neg_l_w"].T, g["neg_r_w"].T, g["neg_r_b"].reshape(1, 256))


def _kf_call(y1, st1, bng, bnb, pww, pwb):
    return pl.pallas_call(
        _kf_body,
        grid=(NBLK,),
        in_specs=[_row_spec(512), _full_spec((2, 512)),
                  _full_spec((1, 512)), _full_spec((1, 512)),
                  _full_spec((512, 256)), _full_spec((1, 256))],
        out_specs=_row_spec(256),
        out_shape=jax.ShapeDtypeStruct((N, 256), jnp.float32),
    )(y1, st1, bng.reshape(1, 512), bnb.reshape(1, 512),
      pww.T, pwb.reshape(1, 256))


_pool64 = _make_pool_kernel(64)
_pool256 = _make_pool_kernel(256)
_agg1 = _make_agg_kernel(1, 64, split=True)
_agg4 = _make_agg_kernel(4, 128)


def _unpad(x):
    return jnp.concatenate([x[0:K], x[KROWS:KROWS + K]], axis=0)


def _split_agg(agg, nch, cw):
    a = agg.reshape(nch, 2, NPAD, cw)
    sp = a[:, 0, 0:N].transpose(1, 0, 2).reshape(N, nch * cw)
    sn = a[:, 1, 0:N].transpose(1, 0, 2).reshape(N, nch * cw)
    return sp, sn


def kernel(labels, fx, fy, skip0, skip1, skip2, edges_nn, params):
    p = params
    f32 = jnp.float32
    labels_flat = labels.reshape(B * HW)
    pidx = jnp.arange(HW, dtype=jnp.int32)
    c0 = (pidx // H).astype(f32) / (W - 1)
    c1 = (pidx % H).astype(f32) / (H - 1)
    skip0T = skip0.reshape(B, 32, HW).transpose(0, 2, 1)
    skip1T = skip1.reshape(B, 256, HW).transpose(0, 2, 1).reshape(B * HW, 256)
    skip2T = skip2.reshape(B, 256, HW).transpose(0, 2, 1).reshape(B * HW, 256)
    small = jnp.concatenate(
        [jnp.broadcast_to(c0[None, :, None], (B, HW, 1)),
         jnp.broadcast_to(c1[None, :, None], (B, HW, 1)),
         fx.reshape(B, HW, 1), fy.reshape(B, HW, 1),
         skip0T, jnp.zeros((B, HW, 28), f32)], axis=2).reshape(B * HW, 64)
    neginf = jnp.full((K, LANES), -3.0e38, f32)

    pooled_small = _unpad(_pool64(small, labels_flat, neginf))
    pooled_skip1 = _unpad(_pool256(skip1T, labels_flat, neginf))
    pooled_skip2 = _unpad(_pool256(skip2T, labels_flat, neginf))

    epad = EPADDED - E
    src = jnp.concatenate([edges_nn[0], jnp.zeros((epad,), jnp.int32)])
    dst = jnp.concatenate([edges_nn[1], jnp.full((epad,), N, jnp.int32)])
    sgn = jnp.concatenate([edges_nn[2], jnp.ones((epad,), jnp.int32)])
    zeros64 = jnp.zeros(((2 * NPAD) // NSUB, 64), jnp.bfloat16)
    zeros128 = jnp.zeros(((2 * NPAD) // NSUB, 128), jnp.bfloat16)

    xpad, xpadb = pl.pallas_call(
        _kx0_body,
        out_shape=(jax.ShapeDtypeStruct((N, 64), f32),
                   jax.ShapeDtypeStruct((N, 64), jnp.bfloat16)),
    )(pooled_small,
      p["bn_pre_g"].reshape(1, 4), p["bn_pre_b"].reshape(1, 4))

    agg0 = _agg1(xpadb, src, dst, sgn, zeros64)
    sp0 = agg0[0:N]
    sn0 = agg0[NPAD:NPAD + N]
    sp0b = agg0[2 * NPAD:2 * NPAD + N]
    sn0b = agg0[3 * NPAD:3 * NPAD + N]

    y10, invc, st10 = _ky0_call(sp0, sn0, sp0b, sn0b, xpad, p["g0"])
    y20, st20 = _kz_call(y10, st10, pooled_skip1,
                         p["g0"]["bn_g"], p["g0"]["bn_b"],
                         p["m1_pos"], p["m1_neg"])
    xmc1, xcb1 = _kx_call(y20, st20, p["m1_pos"], p["m1_neg"])

    agg1 = _agg4(xcb1.reshape(4 * N, 128), src, dst, sgn, zeros128)
    sp1, sn1 = _split_agg(agg1, 4, 128)

    y11, st11 = _ky_call(sp1, sn1, xmc1, invc, p["g1"])
    y21, st21 = _kz_call(y11, st11, pooled_skip2,
                         p["g1"]["bn_g"], p["g1"]["bn_b"],
                         p["m2_pos"], p["m2_neg"])
    xmc2, xcb2 = _kx_call(y21, st21, p["m2_pos"], p["m2_neg"])

    agg2 = _agg4(xcb2.reshape(4 * N, 128), src, dst, sgn, zeros128)
    sp2, sn2 = _split_agg(agg2, 4, 128)

    y12, st12 = _ky_call(sp2, sn2, xmc2, invc, p["g2"])
    return _kf_call(y12, st12, p["g2"]["bn_g"], p["g2"]["bn_b"],
                    p["pw_w"], p["pw_b"])
